# Initial kernel scaffold; baseline (speedup 1.0000x reference)
import jax
import jax.numpy as jnp
from jax.experimental import pallas as pl


def kernel(x, edge_index, W1, b1, W2, b2, Wlin, blin):
    raise NotImplementedError("write your pallas kernel here")


# speedup vs baseline: 14.5373x; 14.5373x over previous
import functools

import jax
import jax.numpy as jnp
from jax import lax
from jax.experimental import pallas as pl
from jax.experimental.pallas import tpu as pltpu
from jax.experimental.pallas import tpu_sc as plsc

N = 10000
E = 160000
D = 256
H = 128
NC = 2
NS = 16
CHUNK = 64
CH = 160
EP = NS * CH * CHUNK
NP_ = 10240
RPT = NP_ // NS
BM = 640
NBLK = NP_ // BM


def _deg_body(dst_hbm, zero_hbm, out_hbm, dst_v, ones_v, acc_sh):
    cid = lax.axis_index("c")
    sid = lax.axis_index("s")

    def fill_ones(t, carry):
        ones_v[t, :] = jnp.full((16,), 1.0, jnp.float32)
        return carry

    lax.fori_loop(0, CHUNK, fill_ones, 0)

    pltpu.sync_copy(zero_hbm.at[pl.ds(sid * RPT, RPT)],
                    acc_sh.at[pl.ds(sid * RPT, RPT)])
    plsc.subcore_barrier()

    def chunk(j, carry):
        pltpu.sync_copy(ones_v, acc_sh.at[dst_v.at[j]], add=True)
        return carry

    for half in range(2):
        pltpu.sync_copy(dst_hbm.at[sid, half], dst_v)
        lax.fori_loop(0, CH // 2, chunk, 0)
    plsc.subcore_barrier()
    pltpu.sync_copy(acc_sh.at[pl.ds(sid * RPT, RPT)],
                    out_hbm.at[cid, pl.ds(sid * RPT, RPT)])


def _scatter_body(tab_hbm, src_hbm, dst_hbm, out_hbm,
                  src_v, dst_v, r0, r1, acc_sh, s0, s1):
    cid = lax.axis_index("c")
    sid = lax.axis_index("s")
    bufs = (r0, r1)
    sems = (s0, s1)

    pltpu.sync_copy(tab_hbm.at[pl.ds(cid * NP_ + sid * RPT, RPT)],
                    acc_sh.at[pl.ds(sid * RPT, RPT)])
    plsc.subcore_barrier()

    HC = CH // 2

    def step(j, carry):
        for b in range(2):
            c = j * 2 + b
            pltpu.make_async_copy(tab_hbm.at[src_v.at[c]],
                                  bufs[b], sems[b]).wait()
            pltpu.sync_copy(bufs[b], acc_sh.at[dst_v.at[c]], add=True)
            pltpu.async_copy(tab_hbm.at[src_v.at[c + 2]], bufs[b], sems[b])
        return carry

    for half in range(2):
        pltpu.sync_copy(src_hbm.at[cid, sid, half], src_v)
        pltpu.sync_copy(dst_hbm.at[sid, half], dst_v)
        for b in range(2):
            pltpu.async_copy(tab_hbm.at[src_v.at[b]], bufs[b], sems[b])
        lax.fori_loop(0, HC // 2 - 1, step, 0)
        for b in range(2):
            c = HC - 2 + b
            pltpu.make_async_copy(tab_hbm.at[src_v.at[c]],
                                  bufs[b], sems[b]).wait()
            pltpu.sync_copy(bufs[b], acc_sh.at[dst_v.at[c]], add=True)

    plsc.subcore_barrier()
    pltpu.sync_copy(acc_sh.at[pl.ds(sid * RPT, RPT)],
                    out_hbm.at[pl.ds(cid * NP_ + sid * RPT, RPT)])


@functools.cache
def _sc_kernels():
    mesh = plsc.VectorSubcoreMesh(core_axis_name="c", subcore_axis_name="s",
                                  num_cores=NC, num_subcores=NS)
    deg = pl.kernel(
        _deg_body,
        out_type=jax.ShapeDtypeStruct((NC, NP_, 16), jnp.float32),
        mesh=mesh,
        scratch_types=[
            pltpu.VMEM((CH // 2, CHUNK), jnp.int32),
            pltpu.VMEM((CHUNK, 16), jnp.float32),
            pltpu.VMEM_SHARED((NP_, 16), jnp.float32),
        ],
    )
    scat = pl.kernel(
        _scatter_body,
        out_type=jax.ShapeDtypeStruct((NC * NP_, H), jnp.float32),
        mesh=mesh,
        scratch_types=[
            pltpu.VMEM((CH // 2, CHUNK), jnp.int32),
            pltpu.VMEM((CH // 2, CHUNK), jnp.int32),
            pltpu.VMEM((CHUNK, H), jnp.float32),
            pltpu.VMEM((CHUNK, H), jnp.float32),
            pltpu.VMEM_SHARED((NP_, H), jnp.float32),
            pltpu.SemaphoreType.DMA,
            pltpu.SemaphoreType.DMA,
        ],
    )
    return deg, scat


def _tck1_body(p_ref, x_ref, w_ref, o_ref):
    dis = lax.rsqrt(1.0 + p_ref[:, 0:1])
    h = jnp.dot(x_ref[...], w_ref[...], preferred_element_type=jnp.float32)
    o_ref[...] = h * dis


_tck1 = pl.pallas_call(
    _tck1_body,
    grid=(NBLK, NC),
    in_specs=[
        pl.BlockSpec((BM, 16), lambda i, j: (i, 0)),
        pl.BlockSpec((BM, D), lambda i, j: (i, 0)),
        pl.BlockSpec((D, H), lambda i, j: (0, j)),
    ],
    out_specs=pl.BlockSpec((BM, H), lambda i, j: (j * NBLK + i, 0)),
    out_shape=jax.ShapeDtypeStruct((NC * NP_, H), jnp.float32),
)


def _tck2_body(p_ref, a0_ref, a1_ref, b_ref, w_ref, o_ref):
    dis = lax.rsqrt(1.0 + p_ref[:, 0:1])
    agg = jnp.concatenate([a0_ref[...], a1_ref[...]], axis=1)
    hcur = jnp.maximum(agg * dis + b_ref[...], 0.0)
    o_ref[...] = jnp.dot(hcur, w_ref[...],
                         preferred_element_type=jnp.float32) * dis


_tck2 = pl.pallas_call(
    _tck2_body,
    grid=(NBLK, NC),
    in_specs=[
        pl.BlockSpec((BM, 16), lambda i, j: (i, 0)),
        pl.BlockSpec((BM, H), lambda i, j: (i, 0)),
        pl.BlockSpec((BM, H), lambda i, j: (NBLK + i, 0)),
        pl.BlockSpec((1, D), lambda i, j: (0, 0)),
        pl.BlockSpec((D, H), lambda i, j: (0, j)),
    ],
    out_specs=pl.BlockSpec((BM, H), lambda i, j: (j * NBLK + i, 0)),
    out_shape=jax.ShapeDtypeStruct((NC * NP_, H), jnp.float32),
)


def _tck3_body(p_ref, a0_ref, a1_ref, b_ref, wl_ref, bl_ref, o_ref, acc_ref):
    i = pl.program_id(0)

    @pl.when(i == 0)
    def _():
        acc_ref[...] = jnp.zeros_like(acc_ref)

    dis = lax.rsqrt(1.0 + p_ref[:, 0:1])
    agg = jnp.concatenate([a0_ref[...], a1_ref[...]], axis=1)
    h2 = jnp.maximum(agg * dis + b_ref[...], 0.0)
    rows = i * BM + lax.broadcasted_iota(jnp.int32, (BM, 1), 0)
    h2 = jnp.where(rows < N, h2, 0.0)
    acc_ref[...] += jnp.sum(h2, axis=0, keepdims=True)

    @pl.when(i == NBLK - 1)
    def _():
        g = acc_ref[...] * (1.0 / N)
        o_ref[...] = jnp.dot(g, wl_ref[...],
                             preferred_element_type=jnp.float32) + bl_ref[...]


_tck3 = pl.pallas_call(
    _tck3_body,
    grid=(NBLK,),
    in_specs=[
        pl.BlockSpec((BM, 16), lambda i: (i, 0)),
        pl.BlockSpec((BM, H), lambda i: (i, 0)),
        pl.BlockSpec((BM, H), lambda i: (NBLK + i, 0)),
        pl.BlockSpec((1, D), lambda i: (0, 0)),
        pl.BlockSpec((D, D), lambda i: (0, 0)),
        pl.BlockSpec((1, D), lambda i: (0, 0)),
    ],
    out_specs=pl.BlockSpec((1, D), lambda i: (0, 0)),
    out_shape=jax.ShapeDtypeStruct((1, D), jnp.float32),
    scratch_shapes=[pltpu.VMEM((1, D), jnp.float32)],
)


def kernel(x, edge_index, W1, b1, W2, b2, Wlin, blin):
    pad = EP - E
    pad_src = jnp.arange(pad, dtype=jnp.int32) % N
    pad_dst = N + jnp.arange(pad, dtype=jnp.int32) % 16
    srcp = jnp.concatenate([edge_index[0], pad_src]).reshape(NS, 2, CH // 2, CHUNK)
    dstp = jnp.concatenate([edge_index[1], pad_dst]).reshape(NS, 2, CH // 2, CHUNK)
    src2 = jnp.stack([srcp, srcp + NP_])
    xp = jnp.pad(x, ((0, NP_ - N), (0, 0)))

    deg_kernel, scatter_kernel = _sc_kernels()
    degp = deg_kernel(dstp, jnp.zeros((NP_, 16), jnp.float32))
    p0 = degp[0]
    h1t = _tck1(p0, xp, W1)
    agg1 = scatter_kernel(h1t, src2, dstp)
    h2t = _tck2(p0, agg1, agg1, b1.reshape(1, D), W2)
    agg2 = scatter_kernel(h2t, src2, dstp)
    return _tck3(p0, agg2, agg2, b2.reshape(1, D), Wlin, blin.reshape(1, D))


# speedup vs baseline: 16.8502x; 1.1591x over previous
import functools

import jax
import jax.numpy as jnp
from jax import lax
from jax.experimental import pallas as pl
from jax.experimental.pallas import tpu as pltpu
from jax.experimental.pallas import tpu_sc as plsc

N = 10000
E = 160000
D = 256
H = 128
NC = 2
NS = 16
CHUNK = 64
NST = 4
SC_CH = 40
CH = NST * SC_CH
EP = NS * CH * CHUNK
NP_ = 10240
RPT = NP_ // NS
ACC_ROWS = 10112
RPA = ACC_ROWS // NS
BM = 640
NBLK = NP_ // BM


def _deg_body(dst_hbm, zero_hbm, out_hbm, dst_v, ones_v, acc_sh):
    cid = lax.axis_index("c")
    sid = lax.axis_index("s")

    def fill_ones(t, carry):
        ones_v[t, :] = jnp.full((16,), 1.0, jnp.float32)
        return carry

    lax.fori_loop(0, CHUNK, fill_ones, 0)

    pltpu.sync_copy(zero_hbm.at[pl.ds(sid * RPT, RPT)],
                    acc_sh.at[pl.ds(sid * RPT, RPT)])
    plsc.subcore_barrier()

    def chunk(j, carry):
        pltpu.sync_copy(ones_v, acc_sh.at[dst_v.at[j]], add=True)
        return carry

    for stage in range(NST):
        pltpu.sync_copy(dst_hbm.at[sid, stage], dst_v)
        lax.fori_loop(0, SC_CH, chunk, 0)
    plsc.subcore_barrier()
    pltpu.sync_copy(acc_sh.at[pl.ds(sid * RPT, RPT)],
                    out_hbm.at[cid, pl.ds(sid * RPT, RPT)])


def _scatter_body(tab_hbm, src_hbm, dst_hbm, out_hbm,
                  src_v, dst_v, r0, r1, r2, r3, acc_sh,
                  g0, g1, g2, g3):
    cid = lax.axis_index("c")
    sid = lax.axis_index("s")
    bufs = (r0, r1, r2, r3)
    gsem = (g0, g1, g2, g3)

    pltpu.sync_copy(tab_hbm.at[pl.ds(cid * NP_ + sid * RPA, RPA)],
                    acc_sh.at[pl.ds(sid * RPA, RPA)])
    plsc.subcore_barrier()

    def gather(c, b):
        pltpu.async_copy(tab_hbm.at[src_v.at[c]], bufs[b], gsem[b])

    def wait_gather(c, b):
        pltpu.make_async_copy(tab_hbm.at[src_v.at[c]], bufs[b], gsem[b]).wait()

    def scatter(c, b):
        pltpu.sync_copy(bufs[b], acc_sh.at[dst_v.at[c]], add=True)

    for stage in range(NST):
        pltpu.sync_copy(src_hbm.at[cid, sid, stage], src_v)
        pltpu.sync_copy(dst_hbm.at[sid, stage], dst_v)
        for b in range(4):
            gather(b, b)

        def group(j, carry):
            for b in range(4):
                c = j * 4 + b
                wait_gather(c, b)
                scatter(c, b)
                gather(c + 4, b)
            return carry

        lax.fori_loop(0, SC_CH // 4 - 1, group, 0)
        for b in range(4):
            c = SC_CH - 4 + b
            wait_gather(c, b)
            scatter(c, b)

    plsc.subcore_barrier()
    pltpu.sync_copy(acc_sh.at[pl.ds(sid * RPA, RPA)],
                    out_hbm.at[pl.ds(cid * NP_ + sid * RPA, RPA)])


@functools.cache
def _sc_kernels():
    mesh = plsc.VectorSubcoreMesh(core_axis_name="c", subcore_axis_name="s",
                                  num_cores=NC, num_subcores=NS)
    deg = pl.kernel(
        _deg_body,
        out_type=jax.ShapeDtypeStruct((NC, NP_, 16), jnp.float32),
        mesh=mesh,
        scratch_types=[
            pltpu.VMEM((SC_CH, CHUNK), jnp.int32),
            pltpu.VMEM((CHUNK, 16), jnp.float32),
            pltpu.VMEM_SHARED((NP_, 16), jnp.float32),
        ],
    )
    scat = pl.kernel(
        _scatter_body,
        out_type=jax.ShapeDtypeStruct((NC * NP_, H), jnp.float32),
        mesh=mesh,
        scratch_types=[
            pltpu.VMEM((SC_CH, CHUNK), jnp.int32),
            pltpu.VMEM((SC_CH, CHUNK), jnp.int32),
            pltpu.VMEM((CHUNK, H), jnp.float32),
            pltpu.VMEM((CHUNK, H), jnp.float32),
            pltpu.VMEM((CHUNK, H), jnp.float32),
            pltpu.VMEM((CHUNK, H), jnp.float32),
            pltpu.VMEM_SHARED((ACC_ROWS, H), jnp.float32),
            pltpu.SemaphoreType.DMA,
            pltpu.SemaphoreType.DMA,
            pltpu.SemaphoreType.DMA,
            pltpu.SemaphoreType.DMA,
        ],
    )
    return deg, scat


def _tck1_body(p_ref, x_ref, w_ref, o_ref):
    dis = lax.rsqrt(1.0 + p_ref[:, 0:1])
    h = jnp.dot(x_ref[...], w_ref[...], preferred_element_type=jnp.float32)
    o_ref[...] = h * dis


_tck1 = pl.pallas_call(
    _tck1_body,
    grid=(NBLK, NC),
    in_specs=[
        pl.BlockSpec((BM, 16), lambda i, j: (i, 0)),
        pl.BlockSpec((BM, D), lambda i, j: (i, 0)),
        pl.BlockSpec((D, H), lambda i, j: (0, j)),
    ],
    out_specs=pl.BlockSpec((BM, H), lambda i, j: (j * NBLK + i, 0)),
    out_shape=jax.ShapeDtypeStruct((NC * NP_, H), jnp.float32),
)


def _tck2_body(p_ref, a0_ref, a1_ref, b_ref, w_ref, o_ref):
    dis = lax.rsqrt(1.0 + p_ref[:, 0:1])
    agg = jnp.concatenate([a0_ref[...], a1_ref[...]], axis=1)
    hcur = jnp.maximum(agg * dis + b_ref[...], 0.0)
    o_ref[...] = jnp.dot(hcur, w_ref[...],
                         preferred_element_type=jnp.float32) * dis


_tck2 = pl.pallas_call(
    _tck2_body,
    grid=(NBLK, NC),
    in_specs=[
        pl.BlockSpec((BM, 16), lambda i, j: (i, 0)),
        pl.BlockSpec((BM, H), lambda i, j: (i, 0)),
        pl.BlockSpec((BM, H), lambda i, j: (NBLK + i, 0)),
        pl.BlockSpec((1, D), lambda i, j: (0, 0)),
        pl.BlockSpec((D, H), lambda i, j: (0, j)),
    ],
    out_specs=pl.BlockSpec((BM, H), lambda i, j: (j * NBLK + i, 0)),
    out_shape=jax.ShapeDtypeStruct((NC * NP_, H), jnp.float32),
)


def _tck3_body(p_ref, a0_ref, a1_ref, b_ref, wl_ref, bl_ref, o_ref, acc_ref):
    i = pl.program_id(0)

    @pl.when(i == 0)
    def _():
        acc_ref[...] = jnp.zeros_like(acc_ref)

    dis = lax.rsqrt(1.0 + p_ref[:, 0:1])
    agg = jnp.concatenate([a0_ref[...], a1_ref[...]], axis=1)
    h2 = jnp.maximum(agg * dis + b_ref[...], 0.0)
    rows = i * BM + lax.broadcasted_iota(jnp.int32, (BM, 1), 0)
    h2 = jnp.where(rows < N, h2, 0.0)
    acc_ref[...] += jnp.sum(h2, axis=0, keepdims=True)

    @pl.when(i == NBLK - 1)
    def _():
        g = acc_ref[...] * (1.0 / N)
        o_ref[...] = jnp.dot(g, wl_ref[...],
                             preferred_element_type=jnp.float32) + bl_ref[...]


_tck3 = pl.pallas_call(
    _tck3_body,
    grid=(NBLK,),
    in_specs=[
        pl.BlockSpec((BM, 16), lambda i: (i, 0)),
        pl.BlockSpec((BM, H), lambda i: (i, 0)),
        pl.BlockSpec((BM, H), lambda i: (NBLK + i, 0)),
        pl.BlockSpec((1, D), lambda i: (0, 0)),
        pl.BlockSpec((D, D), lambda i: (0, 0)),
        pl.BlockSpec((1, D), lambda i: (0, 0)),
    ],
    out_specs=pl.BlockSpec((1, D), lambda i: (0, 0)),
    out_shape=jax.ShapeDtypeStruct((1, D), jnp.float32),
    scratch_shapes=[pltpu.VMEM((1, D), jnp.float32)],
)


def kernel(x, edge_index, W1, b1, W2, b2, Wlin, blin):
    pad = EP - E
    pad_src = jnp.arange(pad, dtype=jnp.int32) % N
    pad_dst = N + jnp.arange(pad, dtype=jnp.int32) % 16
    srcp = jnp.concatenate([edge_index[0], pad_src]).reshape(NS, NST, SC_CH, CHUNK)
    dstp = jnp.concatenate([edge_index[1], pad_dst]).reshape(NS, NST, SC_CH, CHUNK)
    src2 = jnp.stack([srcp, srcp + NP_])
    xp = jnp.pad(x, ((0, NP_ - N), (0, 0)))

    deg_kernel, scatter_kernel = _sc_kernels()
    degp = deg_kernel(dstp, jnp.zeros((NP_, 16), jnp.float32))
    p0 = degp[0]
    h1t = _tck1(p0, xp, W1)
    agg1 = scatter_kernel(h1t, src2, dstp)
    h2t = _tck2(p0, agg1, agg1, b1.reshape(1, D), W2)
    agg2 = scatter_kernel(h2t, src2, dstp)
    return _tck3(p0, agg2, agg2, b2.reshape(1, D), Wlin, blin.reshape(1, D))


# speedup vs baseline: 18.7730x; 1.1141x over previous
import functools

import jax
import jax.numpy as jnp
from jax import lax
from jax.experimental import pallas as pl
from jax.experimental.pallas import tpu as pltpu
from jax.experimental.pallas import tpu_sc as plsc

N = 10000
E = 160000
D = 256
H = 128
NC = 2
NS = 16
CHUNK = 64
NST = 4
SC_CH = 40
CH = NST * SC_CH
EP = NS * CH * CHUNK
NP_ = 10240
RPT = NP_ // NS
ACC_ROWS = 10112
RPA = ACC_ROWS // NS
BM = 640
NBLK = NP_ // BM


def _deg_body(dst_hbm, zero_hbm, out_hbm, dst_v, ones_v, acc_sh):
    cid = lax.axis_index("c")
    sid = lax.axis_index("s")

    def fill_ones(t, carry):
        ones_v[t, :] = jnp.full((16,), 1.0, jnp.float32)
        return carry

    lax.fori_loop(0, 128, fill_ones, 0)

    pltpu.sync_copy(zero_hbm.at[pl.ds(sid * RPT, RPT)],
                    acc_sh.at[pl.ds(sid * RPT, RPT)])
    plsc.subcore_barrier()

    def chunk(j, carry):
        pltpu.sync_copy(ones_v, acc_sh.at[dst_v.at[j]], add=True)
        return carry

    pltpu.sync_copy(dst_hbm.at[sid], dst_v)
    lax.fori_loop(0, (CH * CHUNK) // 128, chunk, 0)
    plsc.subcore_barrier()
    pltpu.sync_copy(acc_sh.at[pl.ds(sid * RPT, RPT)],
                    out_hbm.at[cid, pl.ds(sid * RPT, RPT)])


def _scatter_body(tab_hbm, src_hbm, dst_hbm, out_hbm,
                  src_v, dst_v, r0, r1, r2, r3, acc_sh,
                  g0, g1, g2, g3):
    cid = lax.axis_index("c")
    sid = lax.axis_index("s")
    bufs = (r0, r1, r2, r3)
    gsem = (g0, g1, g2, g3)

    pltpu.sync_copy(tab_hbm.at[pl.ds(cid * NP_ + sid * RPA, RPA)],
                    acc_sh.at[pl.ds(sid * RPA, RPA)])
    plsc.subcore_barrier()

    def gather(c, b):
        pltpu.async_copy(tab_hbm.at[src_v.at[c]], bufs[b], gsem[b])

    def wait_gather(c, b):
        pltpu.make_async_copy(tab_hbm.at[src_v.at[c]], bufs[b], gsem[b]).wait()

    def scatter(c, b):
        pltpu.sync_copy(bufs[b], acc_sh.at[dst_v.at[c]], add=True)

    for stage in range(NST):
        pltpu.sync_copy(src_hbm.at[cid, sid, stage], src_v)
        pltpu.sync_copy(dst_hbm.at[sid, stage], dst_v)
        for b in range(4):
            gather(b, b)

        def group(j, carry):
            for b in range(4):
                c = j * 4 + b
                wait_gather(c, b)
                scatter(c, b)
                gather(c + 4, b)
            return carry

        lax.fori_loop(0, SC_CH // 4 - 1, group, 0)
        for b in range(4):
            c = SC_CH - 4 + b
            wait_gather(c, b)
            scatter(c, b)

    plsc.subcore_barrier()
    pltpu.sync_copy(acc_sh.at[pl.ds(sid * RPA, RPA)],
                    out_hbm.at[pl.ds(cid * NP_ + sid * RPA, RPA)])


@functools.cache
def _sc_kernels():
    mesh = plsc.VectorSubcoreMesh(core_axis_name="c", subcore_axis_name="s",
                                  num_cores=NC, num_subcores=NS)
    deg = pl.kernel(
        _deg_body,
        out_type=jax.ShapeDtypeStruct((NC, NP_, 16), jnp.float32),
        mesh=mesh,
        scratch_types=[
            pltpu.VMEM(((CH * CHUNK) // 128, 128), jnp.int32),
            pltpu.VMEM((128, 16), jnp.float32),
            pltpu.VMEM_SHARED((NP_, 16), jnp.float32),
        ],
    )
    scat = pl.kernel(
        _scatter_body,
        out_type=jax.ShapeDtypeStruct((NC * NP_, H), jnp.float32),
        mesh=mesh,
        scratch_types=[
            pltpu.VMEM((SC_CH, CHUNK), jnp.int32),
            pltpu.VMEM((SC_CH, CHUNK), jnp.int32),
            pltpu.VMEM((CHUNK, H), jnp.float32),
            pltpu.VMEM((CHUNK, H), jnp.float32),
            pltpu.VMEM((CHUNK, H), jnp.float32),
            pltpu.VMEM((CHUNK, H), jnp.float32),
            pltpu.VMEM_SHARED((ACC_ROWS, H), jnp.float32),
            pltpu.SemaphoreType.DMA,
            pltpu.SemaphoreType.DMA,
            pltpu.SemaphoreType.DMA,
            pltpu.SemaphoreType.DMA,
        ],
    )
    return deg, scat


def _tck1_body(p_ref, x_ref, w_ref, o_ref):
    dis = lax.rsqrt(1.0 + p_ref[:, 0:1])
    h = jnp.dot(x_ref[...], w_ref[...],
                preferred_element_type=jnp.float32) * dis
    o_ref[0] = h[:, :H]
    o_ref[1] = h[:, H:]


_tck1 = pl.pallas_call(
    _tck1_body,
    grid=(NBLK,),
    in_specs=[
        pl.BlockSpec((BM, 16), lambda i: (i, 0)),
        pl.BlockSpec((BM, D), lambda i: (i, 0)),
        pl.BlockSpec((D, D), lambda i: (0, 0)),
    ],
    out_specs=pl.BlockSpec((NC, BM, H), lambda i: (0, i, 0)),
    out_shape=jax.ShapeDtypeStruct((NC, NP_, H), jnp.float32),
)


def _tck2_body(p_ref, a_ref, b_ref, w_ref, o_ref):
    dis = lax.rsqrt(1.0 + p_ref[:, 0:1])
    agg = jnp.concatenate([a_ref[0], a_ref[1]], axis=1)
    hcur = jnp.maximum(agg * dis + b_ref[...], 0.0)
    h = jnp.dot(hcur, w_ref[...],
                preferred_element_type=jnp.float32) * dis
    o_ref[0] = h[:, :H]
    o_ref[1] = h[:, H:]


_tck2 = pl.pallas_call(
    _tck2_body,
    grid=(NBLK,),
    in_specs=[
        pl.BlockSpec((BM, 16), lambda i: (i, 0)),
        pl.BlockSpec((NC, BM, H), lambda i: (0, i, 0)),
        pl.BlockSpec((1, D), lambda i: (0, 0)),
        pl.BlockSpec((D, D), lambda i: (0, 0)),
    ],
    out_specs=pl.BlockSpec((NC, BM, H), lambda i: (0, i, 0)),
    out_shape=jax.ShapeDtypeStruct((NC, NP_, H), jnp.float32),
)


def _tck3_body(p_ref, a_ref, b_ref, wl_ref, bl_ref, o_ref, acc_ref):
    i = pl.program_id(0)

    @pl.when(i == 0)
    def _():
        acc_ref[...] = jnp.zeros_like(acc_ref)

    dis = lax.rsqrt(1.0 + p_ref[:, 0:1])
    agg = jnp.concatenate([a_ref[0], a_ref[1]], axis=1)
    h2 = jnp.maximum(agg * dis + b_ref[...], 0.0)
    rows = i * BM + lax.broadcasted_iota(jnp.int32, (BM, 1), 0)
    h2 = jnp.where(rows < N, h2, 0.0)
    acc_ref[...] += jnp.sum(h2, axis=0, keepdims=True)

    @pl.when(i == NBLK - 1)
    def _():
        g = acc_ref[...] * (1.0 / N)
        o_ref[...] = jnp.dot(g, wl_ref[...],
                             preferred_element_type=jnp.float32) + bl_ref[...]


_tck3 = pl.pallas_call(
    _tck3_body,
    grid=(NBLK,),
    in_specs=[
        pl.BlockSpec((BM, 16), lambda i: (i, 0)),
        pl.BlockSpec((NC, BM, H), lambda i: (0, i, 0)),
        pl.BlockSpec((1, D), lambda i: (0, 0)),
        pl.BlockSpec((D, D), lambda i: (0, 0)),
        pl.BlockSpec((1, D), lambda i: (0, 0)),
    ],
    out_specs=pl.BlockSpec((1, D), lambda i: (0, 0)),
    out_shape=jax.ShapeDtypeStruct((1, D), jnp.float32),
    scratch_shapes=[pltpu.VMEM((1, D), jnp.float32)],
)


def kernel(x, edge_index, W1, b1, W2, b2, Wlin, blin):
    pad = EP - E
    pad_src = jnp.arange(pad, dtype=jnp.int32) % N
    pad_dst = N + jnp.arange(pad, dtype=jnp.int32) % 16
    srcf = jnp.concatenate([edge_index[0], pad_src])
    dstf = jnp.concatenate([edge_index[1], pad_dst])
    srcp = srcf.reshape(NS, NST, SC_CH, CHUNK)
    dstp = dstf.reshape(NS, NST, SC_CH, CHUNK)
    dstp128 = dstf.reshape(NS, (CH * CHUNK) // 128, 128)
    src2 = jnp.stack([srcp, srcp + NP_])
    xp = jnp.pad(x, ((0, NP_ - N), (0, 0)))

    deg_kernel, scatter_kernel = _sc_kernels()
    degp = deg_kernel(dstp128, jnp.zeros((NP_, 16), jnp.float32))
    p0 = degp[0]
    h1t = _tck1(p0, xp, W1)
    agg1 = scatter_kernel(h1t.reshape(NC * NP_, H), src2, dstp)
    h2t = _tck2(p0, agg1.reshape(NC, NP_, H), b1.reshape(1, D), W2)
    agg2 = scatter_kernel(h2t.reshape(NC * NP_, H), src2, dstp)
    return _tck3(p0, agg2.reshape(NC, NP_, H), b2.reshape(1, D), Wlin,
                 blin.reshape(1, D))
